# baseline (speedup 1.0000x reference)
import functools

import jax
import jax.numpy as jnp
from jax import lax
from jax.experimental import pallas as pl
from jax.experimental.pallas import tpu as pltpu
from jax.experimental.pallas import tpu_sc as plsc

N = 10000
D = 128
C = 64
E = 320000

NPAD = 10240
NH = NPAD // 2
BLK = 128
EROWS = E // BLK
NW = 32
WROWS = EROWS // NW
ROWS_PER_TILE = NPAD // 16
DEGW = 16
TCB = 640

_MESH = plsc.VectorSubcoreMesh(core_axis_name="c", subcore_axis_name="s")
_SC_PARAMS = pltpu.CompilerParams(use_tc_tiling_on_sc=False)


def _sc_degree(dst_r):
    @functools.partial(
        pl.kernel,
        mesh=_MESH,
        out_type=jax.ShapeDtypeStruct((2 * NPAD, 64), jnp.float32),
        scratch_types=[
            pltpu.VMEM((WROWS + 1, BLK), jnp.int32),
            pltpu.VMEM((BLK, DEGW), jnp.float32),
            pltpu.VMEM((BLK, DEGW), jnp.float32),
            pltpu.VMEM((ROWS_PER_TILE, DEGW), jnp.float32),
            pltpu.VMEM((ROWS_PER_TILE, 64), jnp.float32),
            pltpu.VMEM_SHARED((NPAD, DEGW), jnp.float32),
            pltpu.SemaphoreType.DMA,
            pltpu.SemaphoreType.DMA,
        ],
        compiler_params=_SC_PARAMS,
    )
    def degk(dst_hbm, out_hbm, didx, obuf, zbuf, r16, r64, acc, isem, zsem):
        c = lax.axis_index("c")
        s = lax.axis_index("s")
        w = c * 16 + s

        ih = pltpu.async_copy(dst_hbm.at[pl.ds(w * WROWS, WROWS)],
                              didx.at[pl.ds(0, WROWS)], isem)

        @pl.loop(0, BLK)
        def _(i):
            zbuf[pl.ds(i, 1), :] = jnp.zeros((1, DEGW), jnp.float32)
            obuf[pl.ds(i, 1), :] = jnp.ones((1, DEGW), jnp.float32)

        for k in range(ROWS_PER_TILE // BLK):
            pltpu.async_copy(
                zbuf, acc.at[pl.ds(s * ROWS_PER_TILE + k * BLK, BLK)], zsem)
        for k in range(ROWS_PER_TILE // BLK):
            pltpu.make_async_copy(
                zbuf, acc.at[pl.ds(s * ROWS_PER_TILE + k * BLK, BLK)],
                zsem).wait()
        ih.wait()
        plsc.subcore_barrier()

        @pl.loop(0, WROWS)
        def _(j):
            pltpu.sync_copy(obuf, acc.at[didx.at[j]], add=True)

        @pl.when(w < 4)
        def _():
            pltpu.sync_copy(dst_hbm.at[pl.ds(NW * WROWS + w, 1)],
                            didx.at[pl.ds(WROWS, 1)])
            pltpu.sync_copy(obuf, acc.at[didx.at[WROWS]], add=True)

        plsc.subcore_barrier()

        off = s * ROWS_PER_TILE
        pltpu.sync_copy(acc.at[pl.ds(off, ROWS_PER_TILE)], r16)

        @pl.loop(0, ROWS_PER_TILE)
        def _(i):
            v = r16[pl.ds(i, 1), :]
            for k in range(4):
                r64[pl.ds(i, 1), pl.ds(DEGW * k, DEGW)] = v

        pltpu.sync_copy(r64, out_hbm.at[pl.ds(c * NPAD + off, ROWS_PER_TILE)])

    return degk(dst_r)


def _sc_hop(t, src_r, dst_r):
    @functools.partial(
        pl.kernel,
        mesh=_MESH,
        out_type=jax.ShapeDtypeStruct((2 * NPAD, C), jnp.float32),
        scratch_types=[
            pltpu.VMEM((WROWS + 1, BLK), jnp.int32),
            pltpu.VMEM((WROWS + 1, BLK), jnp.int32),
            pltpu.VMEM((4, BLK, C), jnp.float32),
            pltpu.VMEM_SHARED((NPAD, C), jnp.float32),
            pltpu.SemaphoreType.DMA,
            pltpu.SemaphoreType.DMA,
            pltpu.SemaphoreType.DMA,
            pltpu.SemaphoreType.DMA,
            pltpu.SemaphoreType.DMA,
            pltpu.SemaphoreType.DMA,
            pltpu.SemaphoreType.DMA,
            pltpu.SemaphoreType.DMA,
        ],
        compiler_params=_SC_PARAMS,
    )
    def hop(t_hbm, src_hbm, dst_hbm, out_hbm,
            sidx, didx, bufs, acc, g0, g1, g2, g3, s0, s1, s2, s3):
        c = lax.axis_index("c")
        s = lax.axis_index("s")
        w = c * 16 + s
        gsem = (g0, g1, g2, g3)
        ssem = (s0, s1, s2, s3)

        def gather(blk, b):
            pltpu.async_copy(t_hbm.at[sidx.at[blk]], bufs.at[b], gsem[b])

        def gwait(b):
            pltpu.make_async_copy(t_hbm.at[sidx.at[0]], bufs.at[b],
                                  gsem[b]).wait()

        def scat(blk, b):
            pltpu.async_copy(bufs.at[b], acc.at[didx.at[blk]], ssem[b],
                             add=True)

        def swait(b):
            pltpu.make_async_copy(bufs.at[b], acc.at[didx.at[0]],
                                  ssem[b]).wait()

        ih1 = pltpu.async_copy(src_hbm.at[pl.ds(w * WROWS, WROWS)],
                               sidx.at[pl.ds(0, WROWS)], g2)
        ih2 = pltpu.async_copy(dst_hbm.at[pl.ds(w * WROWS, WROWS)],
                               didx.at[pl.ds(0, WROWS)], g3)

        @pl.loop(0, BLK)
        def _(i):
            for j in range(C // 16):
                bufs[pl.ds(0, 1), pl.ds(i, 1), pl.ds(16 * j, 16)] = jnp.zeros(
                    (1, 1, 16), jnp.float32)

        for k in range(ROWS_PER_TILE // BLK):
            pltpu.async_copy(
                bufs.at[0], acc.at[pl.ds(s * ROWS_PER_TILE + k * BLK, BLK)],
                s0)
        for k in range(ROWS_PER_TILE // BLK):
            pltpu.make_async_copy(
                bufs.at[0], acc.at[pl.ds(s * ROWS_PER_TILE + k * BLK, BLK)],
                s0).wait()
        ih1.wait()
        ih2.wait()

        gather(0, 0)
        gather(1, 1)
        plsc.subcore_barrier()

        gwait(0); scat(0, 0); gather(2, 2)
        gwait(1); scat(1, 1); gather(3, 3)
        gwait(2); scat(2, 2); swait(0); gather(4, 0)
        gwait(3); scat(3, 3); swait(1); gather(5, 1)

        @pl.loop(4, WROWS - 2, step=4)
        def _(j):
            for b in range(4):
                blk = j + b
                gwait(b)
                scat(blk, b)
                nb = (b + 2) % 4
                swait(nb)
                gather(blk + 2, nb)

        gwait(0); scat(WROWS - 2, 0); swait(2)
        gwait(1); scat(WROWS - 1, 1); swait(3)
        swait(0)
        swait(1)

        @pl.when(w < 4)
        def _():
            pltpu.sync_copy(src_hbm.at[pl.ds(NW * WROWS + w, 1)],
                            sidx.at[pl.ds(WROWS, 1)])
            pltpu.sync_copy(dst_hbm.at[pl.ds(NW * WROWS + w, 1)],
                            didx.at[pl.ds(WROWS, 1)])
            pltpu.sync_copy(t_hbm.at[sidx.at[WROWS]], bufs.at[2])
            pltpu.sync_copy(bufs.at[2], acc.at[didx.at[WROWS]], add=True)

        plsc.subcore_barrier()

        off = s * ROWS_PER_TILE
        pltpu.sync_copy(acc.at[pl.ds(off, ROWS_PER_TILE)],
                        out_hbm.at[pl.ds(c * NPAD + off, ROWS_PER_TILE)])

    return hop(t, src_r, dst_r)


_GRID = NH // TCB


def _dinv(dsum, base):
    deg = dsum + 1.0
    rows = lax.broadcasted_iota(jnp.int32, (dsum.shape[0], 1), 0) + base
    return jnp.where(rows < N // 2, lax.rsqrt(deg), 0.0)


def _tc_input(x3, W, degq):
    def body(x_ref, w_ref, d_ref, o_ref):
        s = _dinv(d_ref[...], pl.program_id(0) * TCB)
        dn = (((1,), (1,)), ((), ()))
        he = lax.dot_general(x_ref[:, 0, :], w_ref[...], dn,
                             preferred_element_type=jnp.float32)
        ho = lax.dot_general(x_ref[:, 1, :], w_ref[...], dn,
                             preferred_element_type=jnp.float32)
        o_ref[...] = jnp.concatenate([he, ho], axis=1) * s

    return pl.pallas_call(
        body,
        grid=(_GRID,),
        in_specs=[
            pl.BlockSpec((TCB, 2, D), lambda i: (i, 0, 0)),
            pl.BlockSpec((C, D), lambda i: (0, 0)),
            pl.BlockSpec((TCB, 128), lambda i: (i, 0)),
        ],
        out_specs=pl.BlockSpec((TCB, 128), lambda i: (i, 0)),
        out_shape=jax.ShapeDtypeStruct((NH, 128), jnp.float32),
    )(x3, W, degq)


def _tc_mid(pv, u2, degq):
    def body(pa_ref, pb_ref, u_ref, d_ref, o_ref):
        s = _dinv(d_ref[...], pl.program_id(0) * TCB)
        cmb = pa_ref[...] + pb_ref[...] + u_ref[...]
        o_ref[...] = cmb * (s * s)

    return pl.pallas_call(
        body,
        grid=(_GRID,),
        in_specs=[
            pl.BlockSpec((TCB, 128), lambda i: (i, 0)),
            pl.BlockSpec((TCB, 128), lambda i: (_GRID + i, 0)),
            pl.BlockSpec((TCB, 128), lambda i: (i, 0)),
            pl.BlockSpec((TCB, 128), lambda i: (i, 0)),
        ],
        out_specs=pl.BlockSpec((TCB, 128), lambda i: (i, 0)),
        out_shape=jax.ShapeDtypeStruct((NH, 128), jnp.float32),
    )(pv, pv, u2, degq)


FBLK = 1000


def _tc_final(qv, qb, w2, degq, bb):
    def body(qa_ref, qb_ref, w_ref, d_ref, b_ref, oe_ref, oo_ref):
        s = _dinv(d_ref[...], pl.program_id(0) * FBLK)
        z = (qa_ref[...] + qb_ref[...] + w_ref[...]) * s + b_ref[...]
        ze = z[:, 0:C]
        zo = z[:, C:128]
        lse_e = jnp.max(ze, axis=1, keepdims=True)
        lse_o = jnp.max(zo, axis=1, keepdims=True)
        lse_e = lse_e + jnp.log(
            jnp.sum(jnp.exp(ze - lse_e), axis=1, keepdims=True))
        lse_o = lse_o + jnp.log(
            jnp.sum(jnp.exp(zo - lse_o), axis=1, keepdims=True))
        oe_ref[...] = ze - lse_e
        oo_ref[...] = zo - lse_o

    return pl.pallas_call(
        body,
        grid=(N // 2 // FBLK,),
        in_specs=[
            pl.BlockSpec((FBLK, 128), lambda i: (i, 0)),
            pl.BlockSpec((FBLK, 128), lambda i: (i, 0)),
            pl.BlockSpec((FBLK, 128), lambda i: (i, 0)),
            pl.BlockSpec((FBLK, 128), lambda i: (i, 0)),
            pl.BlockSpec((1, 128), lambda i: (0, 0)),
        ],
        out_specs=[
            pl.BlockSpec((FBLK, C), lambda i: (i, 0)),
            pl.BlockSpec((FBLK, C), lambda i: (i, 0)),
        ],
        out_shape=[
            jax.ShapeDtypeStruct((N // 2, C), jnp.float32),
            jax.ShapeDtypeStruct((N // 2, C), jnp.float32),
        ],
    )(qv, qb, w2, degq, bb)


def _edge_rows(edge_index):
    if edge_index.dtype == jnp.int64:
        ei = lax.bitcast_convert_type(edge_index, jnp.int32)
        src = ei[0, :, 0] + ei[0, :, 1]
        dst = ei[1, :, 0] + ei[1, :, 1]
    else:
        src = edge_index[0].astype(jnp.int32)
        dst = edge_index[1].astype(jnp.int32)
    return (jnp.reshape(src, (EROWS, BLK)), jnp.reshape(dst, (EROWS, BLK)))


def kernel(x, edge_index, W, b):
    src_r, dst_r = _edge_rows(edge_index)
    x3 = jnp.pad(x, ((0, NPAD - N), (0, 0))).reshape(NH, 2, D)
    bb = jnp.reshape(jnp.concatenate([b, b]), (1, 128))

    deg2 = _sc_degree(dst_r)
    degq = jnp.reshape(deg2[0:NPAD] + deg2[NPAD:2 * NPAD], (NH, 128))
    u2 = _tc_input(x3, W, degq)
    p = _sc_hop(jnp.reshape(u2, (NPAD, C)), src_r, dst_r)
    w2 = _tc_mid(jnp.reshape(p, (NPAD, 128)), u2, degq)
    q = _sc_hop(jnp.reshape(w2, (NPAD, C)), src_r, dst_r)
    qv = jnp.reshape(q, (NPAD, 128))
    qb = lax.slice(qv, (NH, 0), (NH + N // 2, 128))
    oe, oo = _tc_final(qv, qb, w2, degq, bb)
    return jnp.reshape(jnp.stack([oe, oo], axis=1), (N, C))


# speedup vs baseline: 1.1489x; 1.1489x over previous
import functools

import jax
import jax.numpy as jnp
from jax import lax
from jax.experimental import pallas as pl
from jax.experimental.pallas import tpu as pltpu
from jax.experimental.pallas import tpu_sc as plsc

N = 10000
D = 128
C = 64
E = 320000

NPAD = 10240
NH = NPAD // 2
BLK = 128
EROWS = E // BLK
NW = 32
WROWS = EROWS // NW
ROWS_PER_TILE = NPAD // 16
DEGW = 16
TCB = 640

_MESH = plsc.VectorSubcoreMesh(core_axis_name="c", subcore_axis_name="s")
_SC_PARAMS = pltpu.CompilerParams(use_tc_tiling_on_sc=False)


def _sc_degree(dst_r):
    @functools.partial(
        pl.kernel,
        mesh=_MESH,
        out_type=jax.ShapeDtypeStruct((2 * NPAD, 64), jnp.float32),
        scratch_types=[
            pltpu.VMEM((WROWS + 1, BLK), jnp.int32),
            pltpu.VMEM((BLK, DEGW), jnp.float32),
            pltpu.VMEM((BLK, DEGW), jnp.float32),
            pltpu.VMEM((ROWS_PER_TILE, DEGW), jnp.float32),
            pltpu.VMEM((ROWS_PER_TILE, 64), jnp.float32),
            pltpu.VMEM_SHARED((NPAD, DEGW), jnp.float32),
            pltpu.SemaphoreType.DMA,
            pltpu.SemaphoreType.DMA,
        ],
        compiler_params=_SC_PARAMS,
    )
    def degk(dst_hbm, out_hbm, didx, obuf, zbuf, r16, r64, acc, isem, zsem):
        c = lax.axis_index("c")
        s = lax.axis_index("s")
        w = c * 16 + s

        ih = pltpu.async_copy(dst_hbm.at[pl.ds(w * WROWS, WROWS)],
                              didx.at[pl.ds(0, WROWS)], isem)

        @pl.loop(0, BLK)
        def _(i):
            zbuf[pl.ds(i, 1), :] = jnp.zeros((1, DEGW), jnp.float32)
            obuf[pl.ds(i, 1), :] = jnp.ones((1, DEGW), jnp.float32)

        for k in range(ROWS_PER_TILE // BLK):
            pltpu.async_copy(
                zbuf, acc.at[pl.ds(s * ROWS_PER_TILE + k * BLK, BLK)], zsem)
        for k in range(ROWS_PER_TILE // BLK):
            pltpu.make_async_copy(
                zbuf, acc.at[pl.ds(s * ROWS_PER_TILE + k * BLK, BLK)],
                zsem).wait()
        ih.wait()
        plsc.subcore_barrier()

        @pl.loop(0, WROWS)
        def _(j):
            pltpu.sync_copy(obuf, acc.at[didx.at[j]], add=True)

        @pl.when(w < 4)
        def _():
            pltpu.sync_copy(dst_hbm.at[pl.ds(NW * WROWS + w, 1)],
                            didx.at[pl.ds(WROWS, 1)])
            pltpu.sync_copy(obuf, acc.at[didx.at[WROWS]], add=True)

        plsc.subcore_barrier()

        off = s * ROWS_PER_TILE
        pltpu.sync_copy(acc.at[pl.ds(off, ROWS_PER_TILE)], r16)

        @pl.loop(0, ROWS_PER_TILE)
        def _(i):
            v = r16[pl.ds(i, 1), :]
            for k in range(4):
                r64[pl.ds(i, 1), pl.ds(DEGW * k, DEGW)] = v

        pltpu.sync_copy(r64, out_hbm.at[pl.ds(c * NPAD + off, ROWS_PER_TILE)])

    return degk(dst_r)


def _sc_hop(t, src_r, dst_r):
    @functools.partial(
        pl.kernel,
        mesh=_MESH,
        out_type=jax.ShapeDtypeStruct((2 * NPAD, C), jnp.float32),
        scratch_types=[
            pltpu.VMEM((WROWS + 1, BLK), jnp.int32),
            pltpu.VMEM((WROWS + 1, BLK), jnp.int32),
            pltpu.VMEM((4, BLK, C), jnp.float32),
            pltpu.VMEM_SHARED((NPAD, C), jnp.float32),
            pltpu.SemaphoreType.DMA,
            pltpu.SemaphoreType.DMA,
            pltpu.SemaphoreType.DMA,
            pltpu.SemaphoreType.DMA,
            pltpu.SemaphoreType.DMA,
            pltpu.SemaphoreType.DMA,
            pltpu.SemaphoreType.DMA,
            pltpu.SemaphoreType.DMA,
        ],
        compiler_params=_SC_PARAMS,
    )
    def hop(t_hbm, src_hbm, dst_hbm, out_hbm,
            sidx, didx, bufs, acc, g0, g1, g2, g3, s0, s1, s2, s3):
        c = lax.axis_index("c")
        s = lax.axis_index("s")
        w = c * 16 + s
        gsem = (g0, g1, g2, g3)
        ssem = (s0, s1, s2, s3)

        def gather(blk, b):
            pltpu.async_copy(t_hbm.at[sidx.at[blk]], bufs.at[b], gsem[b])

        def gwait(b):
            pltpu.make_async_copy(t_hbm.at[sidx.at[0]], bufs.at[b],
                                  gsem[b]).wait()

        def scat(blk, b):
            pltpu.async_copy(bufs.at[b], acc.at[didx.at[blk]], ssem[b],
                             add=True)

        def swait(b):
            pltpu.make_async_copy(bufs.at[b], acc.at[didx.at[0]],
                                  ssem[b]).wait()

        ih1 = pltpu.async_copy(src_hbm.at[pl.ds(w * WROWS, WROWS)],
                               sidx.at[pl.ds(0, WROWS)], g2)
        ih2 = pltpu.async_copy(dst_hbm.at[pl.ds(w * WROWS, WROWS)],
                               didx.at[pl.ds(0, WROWS)], g3)

        @pl.loop(0, BLK)
        def _(i):
            for j in range(C // 16):
                bufs[pl.ds(0, 1), pl.ds(i, 1), pl.ds(16 * j, 16)] = jnp.zeros(
                    (1, 1, 16), jnp.float32)

        for k in range(ROWS_PER_TILE // BLK):
            pltpu.async_copy(
                bufs.at[0], acc.at[pl.ds(s * ROWS_PER_TILE + k * BLK, BLK)],
                s0)
        for k in range(ROWS_PER_TILE // BLK):
            pltpu.make_async_copy(
                bufs.at[0], acc.at[pl.ds(s * ROWS_PER_TILE + k * BLK, BLK)],
                s0).wait()
        ih1.wait()
        ih2.wait()

        gather(0, 0)
        gather(1, 1)
        plsc.subcore_barrier()

        gwait(0); scat(0, 0); gather(2, 2)
        gwait(1); scat(1, 1); gather(3, 3)
        gwait(2); scat(2, 2); swait(0); gather(4, 0)
        gwait(3); scat(3, 3); swait(1); gather(5, 1)

        @pl.loop(4, WROWS - 2, step=4)
        def _(j):
            for b in range(4):
                blk = j + b
                gwait(b)
                scat(blk, b)
                nb = (b + 2) % 4
                swait(nb)
                gather(blk + 2, nb)

        gwait(0); scat(WROWS - 2, 0); swait(2)
        gwait(1); scat(WROWS - 1, 1); swait(3)
        swait(0)
        swait(1)

        @pl.when(w < 4)
        def _():
            pltpu.sync_copy(src_hbm.at[pl.ds(NW * WROWS + w, 1)],
                            sidx.at[pl.ds(WROWS, 1)])
            pltpu.sync_copy(dst_hbm.at[pl.ds(NW * WROWS + w, 1)],
                            didx.at[pl.ds(WROWS, 1)])
            pltpu.sync_copy(t_hbm.at[sidx.at[WROWS]], bufs.at[2])
            pltpu.sync_copy(bufs.at[2], acc.at[didx.at[WROWS]], add=True)

        plsc.subcore_barrier()

        off = s * ROWS_PER_TILE
        pltpu.sync_copy(acc.at[pl.ds(off, ROWS_PER_TILE)],
                        out_hbm.at[pl.ds(c * NPAD + off, ROWS_PER_TILE)])

    return hop(t, src_r, dst_r)


_GRID = NH // TCB


def _dinv(dsum, base):
    deg = dsum + 1.0
    rows = lax.broadcasted_iota(jnp.int32, (dsum.shape[0], 1), 0) + base
    return jnp.where(rows < N // 2, lax.rsqrt(deg), 0.0)


def _tc_input(x3, W, degq):
    def body(x_ref, w_ref, da_ref, db_ref, o_ref):
        s = _dinv(da_ref[...] + db_ref[...], pl.program_id(0) * TCB)
        dn = (((1,), (1,)), ((), ()))
        he = lax.dot_general(x_ref[:, 0, :], w_ref[...], dn,
                             preferred_element_type=jnp.float32)
        ho = lax.dot_general(x_ref[:, 1, :], w_ref[...], dn,
                             preferred_element_type=jnp.float32)
        o_ref[...] = jnp.concatenate([he, ho], axis=1) * s

    return pl.pallas_call(
        body,
        grid=(_GRID,),
        in_specs=[
            pl.BlockSpec((TCB, 2, D), lambda i: (i, 0, 0)),
            pl.BlockSpec((C, D), lambda i: (0, 0)),
            pl.BlockSpec((TCB, 128), lambda i: (i, 0)),
            pl.BlockSpec((TCB, 128), lambda i: (_GRID + i, 0)),
        ],
        out_specs=pl.BlockSpec((TCB, 128), lambda i: (i, 0)),
        out_shape=jax.ShapeDtypeStruct((NH, 128), jnp.float32),
    )(x3, W, degq, degq)


def _tc_mid(pv, u2, degq):
    def body(pa_ref, pb_ref, u_ref, da_ref, db_ref, o_ref):
        s = _dinv(da_ref[...] + db_ref[...], pl.program_id(0) * TCB)
        cmb = pa_ref[...] + pb_ref[...] + u_ref[...]
        o_ref[...] = cmb * (s * s)

    return pl.pallas_call(
        body,
        grid=(_GRID,),
        in_specs=[
            pl.BlockSpec((TCB, 128), lambda i: (i, 0)),
            pl.BlockSpec((TCB, 128), lambda i: (_GRID + i, 0)),
            pl.BlockSpec((TCB, 128), lambda i: (i, 0)),
            pl.BlockSpec((TCB, 128), lambda i: (i, 0)),
            pl.BlockSpec((TCB, 128), lambda i: (_GRID + i, 0)),
        ],
        out_specs=pl.BlockSpec((TCB, 128), lambda i: (i, 0)),
        out_shape=jax.ShapeDtypeStruct((NH, 128), jnp.float32),
    )(pv, pv, u2, degq, degq)


def _tc_final(qv, w2, degq, bb):
    def body(qa_ref, qb_ref, w_ref, da_ref, db_ref, b_ref, o_ref):
        s = _dinv(da_ref[...] + db_ref[...], pl.program_id(0) * TCB)
        z = (qa_ref[...] + qb_ref[...] + w_ref[...]) * s + b_ref[...]
        ze = z[:, 0:C]
        zo = z[:, C:128]
        lse_e = jnp.max(ze, axis=1, keepdims=True)
        lse_o = jnp.max(zo, axis=1, keepdims=True)
        lse_e = lse_e + jnp.log(
            jnp.sum(jnp.exp(ze - lse_e), axis=1, keepdims=True))
        lse_o = lse_o + jnp.log(
            jnp.sum(jnp.exp(zo - lse_o), axis=1, keepdims=True))
        o_ref[...] = jnp.concatenate([ze - lse_e, zo - lse_o], axis=1)

    return pl.pallas_call(
        body,
        grid=(_GRID,),
        in_specs=[
            pl.BlockSpec((TCB, 128), lambda i: (i, 0)),
            pl.BlockSpec((TCB, 128), lambda i: (_GRID + i, 0)),
            pl.BlockSpec((TCB, 128), lambda i: (i, 0)),
            pl.BlockSpec((TCB, 128), lambda i: (i, 0)),
            pl.BlockSpec((TCB, 128), lambda i: (_GRID + i, 0)),
            pl.BlockSpec((1, 128), lambda i: (0, 0)),
        ],
        out_specs=pl.BlockSpec((TCB, 128), lambda i: (i, 0)),
        out_shape=jax.ShapeDtypeStruct((NH, 128), jnp.float32),
    )(qv, qv, w2, degq, degq, bb)


def _to_rows(col):
    if col.dtype == jnp.int64:
        c32 = lax.bitcast_convert_type(col, jnp.int32)
        col = c32[:, 0] + c32[:, 1]
    return jnp.reshape(col.astype(jnp.int32), (EROWS, BLK))


def kernel(x, edge_index, W, b):
    dst_r = _to_rows(edge_index[1])
    ei_b = lax.optimization_barrier(edge_index)
    src_r = _to_rows(ei_b[0])
    x3 = jnp.pad(x, ((0, NPAD - N), (0, 0))).reshape(NH, 2, D)
    bb = jnp.reshape(jnp.concatenate([b, b]), (1, 128))

    deg2 = _sc_degree(dst_r)
    degq = jnp.reshape(deg2, (2 * NPAD * 64 // 128, 128))
    u2 = _tc_input(x3, W, degq)
    p = _sc_hop(jnp.reshape(u2, (NPAD, C)), src_r, dst_r)
    w2 = _tc_mid(jnp.reshape(p, (NPAD, 128)), u2, degq)
    q = _sc_hop(jnp.reshape(w2, (NPAD, C)), src_r, dst_r)
    out2 = _tc_final(jnp.reshape(q, (NPAD, 128)), w2, degq, bb)
    return jnp.reshape(out2, (NPAD, C))[0:N]
